# baseline (speedup 1.0000x reference)
import functools

import jax
import jax.numpy as jnp
from jax import lax
from jax.experimental import pallas as pl
from jax.experimental.pallas import tpu as pltpu
from jax.experimental.pallas import tpu_sc as plsc

_B = 16384
_F = 26
_D = 64
_H1 = 128
_H2 = 64

_NC = 2
_NS = 16
_NW = _NC * _NS
_BPW = _B // _NW
_CH = 8
_IDX = _CH * _F
_HIDX = _IDX // 2
_NCHUNK = _BPW // _CH
_LANES = 16
_VLIN = 1000016 // 16


def _sc_gather_reduce():
    mesh = plsc.VectorSubcoreMesh(
        core_axis_name="c", subcore_axis_name="s",
        num_cores=_NC, num_subcores=_NS)

    @functools.partial(
        pl.kernel,
        out_type=(
            jax.ShapeDtypeStruct((_B, _D), jnp.float32),
            jax.ShapeDtypeStruct((_B, _D), jnp.float32),
            jax.ShapeDtypeStruct((_B * _F,), jnp.float32),
        ),
        mesh=mesh,
        compiler_params=pltpu.CompilerParams(
            use_tc_tiling_on_sc=False, needs_layout_passes=False),
        scratch_types=[
            pltpu.VMEM((2, 2, _HIDX), jnp.int32),
            pltpu.VMEM((2, 2, _HIDX), jnp.int32),
            pltpu.VMEM((2, _IDX), jnp.int32),
            pltpu.VMEM((2, _IDX, _D), jnp.float32),
            pltpu.VMEM((2, _IDX, _LANES), jnp.float32),
            pltpu.VMEM((2, _IDX), jnp.float32),
            pltpu.VMEM((_CH, _D), jnp.float32),
            pltpu.VMEM((_CH, _D), jnp.float32),
            pltpu.SemaphoreType.DMA,
            pltpu.SemaphoreType.DMA,
        ],
    )
    def sc_fn(xflat_hbm, xhi_hbm, emb_hbm, lin16_hbm, s_hbm, q_hbm, linraw_hbm,
              idx_v, idxhi_v, idxf_v, rows_v, ling_v, linval_v,
              acc_s, acc_q, sem0, sem1):
        wid = lax.axis_index("s") * _NC + lax.axis_index("c")
        row0 = wid * _BPW
        sems = (sem0, sem1)

        def fire(ci, b):
            off = (row0 + ci * _CH) * _F
            pltpu.sync_copy(xflat_hbm.at[pl.ds(off, _IDX)], idxf_v.at[b])
            for h in range(2):
                pltpu.sync_copy(
                    xflat_hbm.at[pl.ds(off + h * _HIDX, _HIDX)], idx_v.at[b, h])
                pltpu.sync_copy(
                    xhi_hbm.at[pl.ds(off + h * _HIDX, _HIDX)], idxhi_v.at[b, h])
                pltpu.async_copy(
                    emb_hbm.at[idx_v.at[b, h]],
                    rows_v.at[b, pl.ds(h * _HIDX, _HIDX)], sems[b])
                pltpu.async_copy(
                    lin16_hbm.at[idxhi_v.at[b, h]],
                    ling_v.at[b, pl.ds(h * _HIDX, _HIDX)], sems[b])

        def drain(b):
            for h in range(2):
                pltpu.make_async_copy(
                    emb_hbm.at[idx_v.at[b, h]],
                    rows_v.at[b, pl.ds(h * _HIDX, _HIDX)], sems[b]).wait()
                pltpu.make_async_copy(
                    lin16_hbm.at[idxhi_v.at[b, h]],
                    ling_v.at[b, pl.ds(h * _HIDX, _HIDX)], sems[b]).wait()

        def reduce_chunk(ci, b):
            for r in range(_CH):
                for db in range(_D // _LANES):
                    sl = pl.ds(db * _LANES, _LANES)
                    v = rows_v[b, r * _F, sl]
                    acc = v
                    accq = v * v
                    for f in range(1, _F):
                        v = rows_v[b, r * _F + f, sl]
                        acc = acc + v
                        accq = accq + v * v
                    acc_s[r, sl] = acc
                    acc_q[r, sl] = accq
            lane_iota = lax.iota(jnp.int32, _LANES)
            for g in range(_IDX // _LANES):
                sl = pl.ds(g * _LANES, _LANES)
                lanes = lax.bitwise_and(idxf_v[b, sl], jnp.int32(15))
                rows16 = lane_iota + jnp.int32(g * _LANES)
                linval_v[b, sl] = plsc.load_gather(
                    ling_v.at[b], [rows16, lanes])
            out_r = row0 + ci * _CH
            pltpu.sync_copy(acc_s, s_hbm.at[pl.ds(out_r, _CH)])
            pltpu.sync_copy(acc_q, q_hbm.at[pl.ds(out_r, _CH)])
            pltpu.sync_copy(linval_v.at[b], linraw_hbm.at[pl.ds(out_r * _F, _IDX)])

        fire(0, 0)

        def outer(k, carry):
            for b in range(2):
                ci = 2 * k + b

                @pl.when(ci + 1 < _NCHUNK)
                def _():
                    fire(ci + 1, 1 - b)

                drain(b)
                reduce_chunk(ci, b)
            return carry

        lax.fori_loop(0, _NCHUNK // 2, outer, 0)

    return sc_fn


_TC_BLK = 2048


def _tc_head(s_ref, q_ref, linr_ref, w1_ref, b1_ref, w2_ref, b2_ref,
             wht_ref, c0_ref, o_ref):
    inter = 0.5 * (s_ref[...] * s_ref[...] - q_ref[...])
    h = jnp.maximum(
        lax.dot_general(inter, w1_ref[...], (((1,), (0,)), ((), ())),
                        precision=lax.Precision.HIGHEST,
                        preferred_element_type=jnp.float32) + b1_ref[...], 0.0)
    h = jnp.maximum(
        lax.dot_general(h, w2_ref[...], (((1,), (0,)), ((), ())),
                        precision=lax.Precision.HIGHEST,
                        preferred_element_type=jnp.float32) + b2_ref[...], 0.0)
    head = jnp.sum(h * wht_ref[...], axis=1)
    lin = jnp.sum(linr_ref[...], axis=1)
    out = head + lin + c0_ref[0, 0]
    o_ref[...] = jnp.clip(out, -2.0, 2.0)


def kernel(x, emb_table, lin_table, bias, W1, b1, W2, b2, Wh, bh):
    xflat = x.reshape(_B * _F).astype(jnp.int32)
    xhi = jax.lax.shift_right_logical(xflat, 4)
    lin16 = jnp.concatenate(
        [lin_table.reshape(-1), jnp.zeros((15,), jnp.float32)]).reshape(_VLIN, 16)
    s, q, linraw = _sc_gather_reduce()(xflat, xhi, emb_table, lin16)
    linr = linraw.reshape(_B, _F)
    c0 = (bias + bh).reshape(1, 1)
    wht = Wh.reshape(1, _H2)

    grid = _B // _TC_BLK
    out = pl.pallas_call(
        _tc_head,
        grid=(grid,),
        in_specs=[
            pl.BlockSpec((_TC_BLK, _D), lambda i: (i, 0)),
            pl.BlockSpec((_TC_BLK, _D), lambda i: (i, 0)),
            pl.BlockSpec((_TC_BLK, _F), lambda i: (i, 0)),
            pl.BlockSpec((_D, _H1), lambda i: (0, 0)),
            pl.BlockSpec((_H1,), lambda i: (0,)),
            pl.BlockSpec((_H1, _H2), lambda i: (0, 0)),
            pl.BlockSpec((_H2,), lambda i: (0,)),
            pl.BlockSpec((1, _H2), lambda i: (0, 0)),
            pl.BlockSpec((1, 1), lambda i: (0, 0)),
        ],
        out_specs=pl.BlockSpec((_TC_BLK,), lambda i: (i,)),
        out_shape=jax.ShapeDtypeStruct((_B,), jnp.float32),
    )(s, q, linr, W1, b1, W2, b2, wht, c0)
    return out
